# baseline (speedup 1.0000x reference)
import functools

import jax
import jax.numpy as jnp
from jax.experimental import pallas as pl
from jax.experimental.pallas import tpu as pltpu

B = 256
A = 32
NUM_LAYERS = 4
EMB = 128
ENC_DIM = 64
NCLS = 4
XE_DIM = 64
PE_DIM = 16

SCENES_PER_BLOCK = 32
CHUNKS = 4


def _fwd_kernel(xe_ref, pe_ref, na_ref, t_ref, w_in_ref, b_in_ref,
                msg_w1_ref, msg_b1_ref, msg_w2_ref, msg_b2_ref, upd_w1_ref,
                upd_b1_ref, upd_w2_ref, upd_b2_ref, fc_w1_ref, fc_b1_ref,
                fc_w2_ref, fc_b2_ref, out_ref, *, scenes):
    rows = scenes * A
    h = (jnp.dot(xe_ref[...], w_in_ref[:XE_DIM],
                 preferred_element_type=jnp.float32)
         + jnp.dot(pe_ref[...], w_in_ref[XE_DIM:XE_DIM + PE_DIM],
                   preferred_element_type=jnp.float32)
         + na_ref[...] * w_in_ref[XE_DIM + PE_DIM]
         + b_in_ref[...])

    t = t_ref[...]
    t1h = (t == jax.lax.broadcasted_iota(jnp.int32, (rows, NCLS), 1))
    t1h = t1h.astype(jnp.float32)

    for l in range(NUM_LAYERS):
        w1 = msg_w1_ref[l]
        pd = (jnp.dot(h, w1[:EMB], preferred_element_type=jnp.float32)
              + jnp.dot(t1h, w1[2 * EMB + NCLS:], preferred_element_type=jnp.float32)
              + msg_b1_ref[l])
        ps = (jnp.dot(h, w1[EMB:2 * EMB], preferred_element_type=jnp.float32)
              + jnp.dot(t1h, w1[2 * EMB:2 * EMB + NCLS], preferred_element_type=jnp.float32))
        m1 = jnp.tanh(ps.reshape(scenes, A, 1, EMB)
                      + pd.reshape(scenes, 1, A, EMB))
        m2 = jnp.tanh(jnp.dot(m1.reshape(scenes * A * A, EMB), msg_w2_ref[l],
                              preferred_element_type=jnp.float32)
                      + msg_b2_ref[l])
        aggr = m2.reshape(scenes, A, A, EMB).sum(axis=1).reshape(rows, EMB)
        wu = upd_w1_ref[l]
        u = jnp.tanh(jnp.dot(h, wu[:EMB], preferred_element_type=jnp.float32)
                     + jnp.dot(aggr, wu[EMB:], preferred_element_type=jnp.float32)
                     + upd_b1_ref[l])
        u = jnp.tanh(jnp.dot(u, upd_w2_ref[l], preferred_element_type=jnp.float32)
                     + upd_b2_ref[l])
        h = h + u

    pooled = h.reshape(scenes, A, EMB).sum(axis=1) * (1.0 / A)
    o = jnp.tanh(jnp.dot(pooled, fc_w1_ref[...], preferred_element_type=jnp.float32)
                 + fc_b1_ref[...])
    out_ref[...] = (jnp.dot(o, fc_w2_ref[...], preferred_element_type=jnp.float32)
                    + fc_b2_ref[...])


def kernel(pos, x_enc, pos_emb, numAgents_emb, T, W_in, b_in, msg_W1, msg_b1,
           msg_W2, msg_b2, upd_W1, upd_b1, upd_W2, upd_b2, fc_W1, fc_b1,
           fc_W2, fc_b2, *, interpret=False):
    del pos
    b, a = T.shape
    n = b * a
    xe = x_enc.reshape(n, XE_DIM)
    pe = pos_emb.reshape(n, PE_DIM)
    na = numAgents_emb.reshape(n, 1)
    t = T.astype(jnp.int32).reshape(n, 1)

    scenes = SCENES_PER_BLOCK
    rows = scenes * a
    grid = (b // scenes,)

    def rowmap(i):
        return (i, 0)

    def fixed2(i):
        return (0, 0)

    def fixed3(i):
        return (0, 0, 0)

    full2 = lambda arr: pl.BlockSpec(arr.shape, fixed2)
    full3 = lambda arr: pl.BlockSpec(arr.shape, fixed3)

    b_in2 = b_in.reshape(1, EMB)
    msg_b1r = msg_b1.reshape(NUM_LAYERS, 1, EMB)
    msg_b2r = msg_b2.reshape(NUM_LAYERS, 1, EMB)
    upd_b1r = upd_b1.reshape(NUM_LAYERS, 1, EMB)
    upd_b2r = upd_b2.reshape(NUM_LAYERS, 1, EMB)
    fc_b1r = fc_b1.reshape(1, EMB // 2)
    fc_b2r = fc_b2.reshape(1, ENC_DIM)

    out = pl.pallas_call(
        functools.partial(_fwd_kernel, scenes=scenes),
        grid=grid,
        in_specs=[
            pl.BlockSpec((rows, XE_DIM), rowmap),
            pl.BlockSpec((rows, PE_DIM), rowmap),
            pl.BlockSpec((rows, 1), rowmap),
            pl.BlockSpec((rows, 1), rowmap),
            full2(W_in), full2(b_in2),
            full3(msg_W1), full3(msg_b1r),
            full3(msg_W2), full3(msg_b2r),
            full3(upd_W1), full3(upd_b1r),
            full3(upd_W2), full3(upd_b2r),
            full2(fc_W1), full2(fc_b1r),
            full2(fc_W2), full2(fc_b2r),
        ],
        out_specs=pl.BlockSpec((scenes, ENC_DIM), rowmap),
        out_shape=jax.ShapeDtypeStruct((b, ENC_DIM), jnp.float32),
        compiler_params=pltpu.CompilerParams(
            dimension_semantics=("parallel",)),
        interpret=interpret,
    )(xe, pe, na, t, W_in, b_in2, msg_W1, msg_b1r, msg_W2, msg_b2r,
      upd_W1, upd_b1r, upd_W2, upd_b2r, fc_W1, fc_b1r, fc_W2, fc_b2r)
    return out


# speedup vs baseline: 1.0107x; 1.0107x over previous
import functools

import jax
import jax.numpy as jnp
from jax.experimental import pallas as pl
from jax.experimental.pallas import tpu as pltpu

B = 256
A = 32
NUM_LAYERS = 4
EMB = 128
ENC_DIM = 64
NCLS = 4
XE_DIM = 64
PE_DIM = 16

SCENES_PER_BLOCK = 32
CHUNKS = 4


def _fwd_kernel(xe_ref, pe_ref, na_ref, t_ref, w_in_ref, b_in_ref,
                msg_w1_ref, msg_b1_ref, msg_w2_ref, msg_b2_ref, upd_w1_ref,
                upd_b1_ref, upd_w2_ref, upd_b2_ref, fc_w1_ref, fc_b1_ref,
                fc_w2_ref, fc_b2_ref, out_ref, *, scenes):
    rows = scenes * A
    xe = xe_ref[...].reshape(rows, XE_DIM)
    pe = pe_ref[...].reshape(rows, PE_DIM)
    na = na_ref[...].reshape(rows, 1)
    h = (jnp.dot(xe, w_in_ref[:XE_DIM], preferred_element_type=jnp.float32)
         + jnp.dot(pe, w_in_ref[XE_DIM:XE_DIM + PE_DIM],
                   preferred_element_type=jnp.float32)
         + na * w_in_ref[XE_DIM + PE_DIM]
         + b_in_ref[...])

    t = t_ref[...]
    t1h = (t == jax.lax.broadcasted_iota(jnp.int32, (rows, NCLS), 1))
    t1h = t1h.astype(jnp.float32)

    for l in range(NUM_LAYERS):
        w1 = msg_w1_ref[l]
        pd = (jnp.dot(h, w1[:EMB], preferred_element_type=jnp.float32)
              + jnp.dot(t1h, w1[2 * EMB + NCLS:], preferred_element_type=jnp.float32)
              + msg_b1_ref[l])
        ps = (jnp.dot(h, w1[EMB:2 * EMB], preferred_element_type=jnp.float32)
              + jnp.dot(t1h, w1[2 * EMB:2 * EMB + NCLS], preferred_element_type=jnp.float32))
        m1 = jnp.tanh(ps.reshape(scenes, A, 1, EMB)
                      + pd.reshape(scenes, 1, A, EMB))
        m2 = jnp.tanh(jnp.dot(m1.reshape(scenes * A * A, EMB), msg_w2_ref[l],
                              preferred_element_type=jnp.float32)
                      + msg_b2_ref[l])
        aggr = m2.reshape(scenes, A, A, EMB).sum(axis=1).reshape(rows, EMB)
        wu = upd_w1_ref[l]
        u = jnp.tanh(jnp.dot(h, wu[:EMB], preferred_element_type=jnp.float32)
                     + jnp.dot(aggr, wu[EMB:], preferred_element_type=jnp.float32)
                     + upd_b1_ref[l])
        u = jnp.tanh(jnp.dot(u, upd_w2_ref[l], preferred_element_type=jnp.float32)
                     + upd_b2_ref[l])
        h = h + u

    pooled = h.reshape(scenes, A, EMB).sum(axis=1) * (1.0 / A)
    o = jnp.tanh(jnp.dot(pooled, fc_w1_ref[...], preferred_element_type=jnp.float32)
                 + fc_b1_ref[...])
    out_ref[...] = (jnp.dot(o, fc_w2_ref[...], preferred_element_type=jnp.float32)
                    + fc_b2_ref[...])


def kernel(pos, x_enc, pos_emb, numAgents_emb, T, W_in, b_in, msg_W1, msg_b1,
           msg_W2, msg_b2, upd_W1, upd_b1, upd_W2, upd_b2, fc_W1, fc_b1,
           fc_W2, fc_b2, *, interpret=False):
    del pos
    b, a = T.shape
    na = numAgents_emb.reshape(b, a, 1)
    t = T.astype(jnp.int32).reshape(b * a, 1)

    scenes = SCENES_PER_BLOCK
    rows = scenes * a
    grid = (b // scenes,)

    def rowmap(i):
        return (i, 0)

    def rowmap3(i):
        return (i, 0, 0)

    def fixed2(i):
        return (0, 0)

    def fixed3(i):
        return (0, 0, 0)

    full2 = lambda arr: pl.BlockSpec(arr.shape, fixed2)
    full3 = lambda arr: pl.BlockSpec(arr.shape, fixed3)

    b_in2 = b_in.reshape(1, EMB)
    msg_b1r = msg_b1.reshape(NUM_LAYERS, 1, EMB)
    msg_b2r = msg_b2.reshape(NUM_LAYERS, 1, EMB)
    upd_b1r = upd_b1.reshape(NUM_LAYERS, 1, EMB)
    upd_b2r = upd_b2.reshape(NUM_LAYERS, 1, EMB)
    fc_b1r = fc_b1.reshape(1, EMB // 2)
    fc_b2r = fc_b2.reshape(1, ENC_DIM)

    out = pl.pallas_call(
        functools.partial(_fwd_kernel, scenes=scenes),
        grid=grid,
        in_specs=[
            pl.BlockSpec((scenes, a, XE_DIM), rowmap3),
            pl.BlockSpec((scenes, a, PE_DIM), rowmap3),
            pl.BlockSpec((scenes, a, 1), rowmap3),
            pl.BlockSpec((rows, 1), rowmap),
            full2(W_in), full2(b_in2),
            full3(msg_W1), full3(msg_b1r),
            full3(msg_W2), full3(msg_b2r),
            full3(upd_W1), full3(upd_b1r),
            full3(upd_W2), full3(upd_b2r),
            full2(fc_W1), full2(fc_b1r),
            full2(fc_W2), full2(fc_b2r),
        ],
        out_specs=pl.BlockSpec((scenes, ENC_DIM), rowmap),
        out_shape=jax.ShapeDtypeStruct((b, ENC_DIM), jnp.float32),
        compiler_params=pltpu.CompilerParams(
            dimension_semantics=("parallel",)),
        interpret=interpret,
    )(x_enc, pos_emb, na, t, W_in, b_in2, msg_W1, msg_b1r, msg_W2, msg_b2r,
      upd_W1, upd_b1r, upd_W2, upd_b2r, fc_W1, fc_b1r, fc_W2, fc_b2r)
    return out


# speedup vs baseline: 1.0300x; 1.0191x over previous
import functools

import jax
import jax.numpy as jnp
from jax.experimental import pallas as pl
from jax.experimental.pallas import tpu as pltpu

B = 256
A = 32
NUM_LAYERS = 4
EMB = 128
ENC_DIM = 64
NCLS = 4
XE_DIM = 64
PE_DIM = 16

SCENES_PER_BLOCK = 32
CHUNKS = 4


def _fwd_kernel(xe_ref, pena_ref, t_ref, w_in_ref, b_in_ref,
                msg_w1_ref, msg_b1_ref, msg_w2_ref, msg_b2_ref, upd_w1_ref,
                upd_b1_ref, upd_w2_ref, upd_b2_ref, fc_w1_ref, fc_b1_ref,
                fc_w2_ref, fc_b2_ref, out_ref, *, scenes):
    rows = scenes * A
    xe = xe_ref[...].reshape(rows, XE_DIM)
    pena = pena_ref[...]
    h = (jnp.dot(xe, w_in_ref[:XE_DIM], preferred_element_type=jnp.float32)
         + jnp.dot(pena, w_in_ref[XE_DIM:],
                   preferred_element_type=jnp.float32)
         + b_in_ref[...])

    t = t_ref[...]
    t1h = (t == jax.lax.broadcasted_iota(jnp.int32, (rows, NCLS), 1))
    t1h = t1h.astype(jnp.float32)

    for l in range(NUM_LAYERS):
        w1 = msg_w1_ref[l]
        pd = (jnp.dot(h, w1[:EMB], preferred_element_type=jnp.float32)
              + jnp.dot(t1h, w1[2 * EMB + NCLS:], preferred_element_type=jnp.float32)
              + msg_b1_ref[l])
        ps = (jnp.dot(h, w1[EMB:2 * EMB], preferred_element_type=jnp.float32)
              + jnp.dot(t1h, w1[2 * EMB:2 * EMB + NCLS], preferred_element_type=jnp.float32))
        m1 = jnp.tanh(ps.reshape(scenes, A, 1, EMB)
                      + pd.reshape(scenes, 1, A, EMB))
        m2 = jnp.tanh(jnp.dot(m1.reshape(scenes * A * A, EMB), msg_w2_ref[l],
                              preferred_element_type=jnp.float32)
                      + msg_b2_ref[l])
        aggr = m2.reshape(scenes, A, A, EMB).sum(axis=1).reshape(rows, EMB)
        wu = upd_w1_ref[l]
        u = jnp.tanh(jnp.dot(h, wu[:EMB], preferred_element_type=jnp.float32)
                     + jnp.dot(aggr, wu[EMB:], preferred_element_type=jnp.float32)
                     + upd_b1_ref[l])
        u = jnp.tanh(jnp.dot(u, upd_w2_ref[l], preferred_element_type=jnp.float32)
                     + upd_b2_ref[l])
        h = h + u

    pooled = h.reshape(scenes, A, EMB).sum(axis=1) * (1.0 / A)
    o = jnp.tanh(jnp.dot(pooled, fc_w1_ref[...], preferred_element_type=jnp.float32)
                 + fc_b1_ref[...])
    out_ref[...] = (jnp.dot(o, fc_w2_ref[...], preferred_element_type=jnp.float32)
                    + fc_b2_ref[...])


def kernel(pos, x_enc, pos_emb, numAgents_emb, T, W_in, b_in, msg_W1, msg_b1,
           msg_W2, msg_b2, upd_W1, upd_b1, upd_W2, upd_b2, fc_W1, fc_b1,
           fc_W2, fc_b2, *, interpret=False):
    del pos
    b, a = T.shape
    pena = jnp.concatenate([pos_emb, numAgents_emb],
                           axis=-1).reshape(b * a, PE_DIM + 1)
    t = T.astype(jnp.int32).reshape(b * a, 1)

    scenes = SCENES_PER_BLOCK
    rows = scenes * a
    grid = (b // scenes,)

    def rowmap(i):
        return (i, 0)

    def rowmap3(i):
        return (i, 0, 0)

    def fixed2(i):
        return (0, 0)

    def fixed3(i):
        return (0, 0, 0)

    full2 = lambda arr: pl.BlockSpec(arr.shape, fixed2)
    full3 = lambda arr: pl.BlockSpec(arr.shape, fixed3)

    b_in2 = b_in.reshape(1, EMB)
    msg_b1r = msg_b1.reshape(NUM_LAYERS, 1, EMB)
    msg_b2r = msg_b2.reshape(NUM_LAYERS, 1, EMB)
    upd_b1r = upd_b1.reshape(NUM_LAYERS, 1, EMB)
    upd_b2r = upd_b2.reshape(NUM_LAYERS, 1, EMB)
    fc_b1r = fc_b1.reshape(1, EMB // 2)
    fc_b2r = fc_b2.reshape(1, ENC_DIM)

    out = pl.pallas_call(
        functools.partial(_fwd_kernel, scenes=scenes),
        grid=grid,
        in_specs=[
            pl.BlockSpec((scenes, a, XE_DIM), rowmap3),
            pl.BlockSpec((rows, PE_DIM + 1), rowmap),
            pl.BlockSpec((rows, 1), rowmap),
            full2(W_in), full2(b_in2),
            full3(msg_W1), full3(msg_b1r),
            full3(msg_W2), full3(msg_b2r),
            full3(upd_W1), full3(upd_b1r),
            full3(upd_W2), full3(upd_b2r),
            full2(fc_W1), full2(fc_b1r),
            full2(fc_W2), full2(fc_b2r),
        ],
        out_specs=pl.BlockSpec((scenes, ENC_DIM), rowmap),
        out_shape=jax.ShapeDtypeStruct((b, ENC_DIM), jnp.float32),
        compiler_params=pltpu.CompilerParams(
            dimension_semantics=("parallel",)),
        interpret=interpret,
    )(x_enc, pena, t, W_in, b_in2, msg_W1, msg_b1r, msg_W2, msg_b2r,
      upd_W1, upd_b1r, upd_W2, upd_b2r, fc_W1, fc_b1r, fc_W2, fc_b2r)
    return out


# speedup vs baseline: 1.0681x; 1.0370x over previous
import functools

import jax
import jax.numpy as jnp
from jax.experimental import pallas as pl
from jax.experimental.pallas import tpu as pltpu

B = 256
A = 32
NUM_LAYERS = 4
EMB = 128
IN_DIM = 81
ENC_DIM = 64
NCLS = 4

SCENES_PER_BLOCK = 32


def _fwd_kernel(x_ref, t_ref, w_in_ref, b_in_ref, msg_w1_ref, msg_b1_ref,
                msg_w2_ref, msg_b2_ref, upd_w1_ref, upd_b1_ref, upd_w2_ref,
                upd_b2_ref, fc_w1_ref, fc_b1_ref, fc_w2_ref, fc_b2_ref,
                out_ref, *, scenes):
    rows = scenes * A
    x = x_ref[...]
    h = jnp.dot(x, w_in_ref[...], preferred_element_type=jnp.float32)
    h = h + b_in_ref[...]

    t = t_ref[...]
    t1h = (t == jax.lax.broadcasted_iota(jnp.int32, (rows, NCLS), 1))
    t1h = t1h.astype(jnp.float32)

    for l in range(NUM_LAYERS):
        w1 = msg_w1_ref[l]
        pd = (jnp.dot(h, w1[:EMB], preferred_element_type=jnp.float32)
              + jnp.dot(t1h, w1[2 * EMB + NCLS:], preferred_element_type=jnp.float32)
              + msg_b1_ref[l])
        ps = (jnp.dot(h, w1[EMB:2 * EMB], preferred_element_type=jnp.float32)
              + jnp.dot(t1h, w1[2 * EMB:2 * EMB + NCLS], preferred_element_type=jnp.float32))
        m1 = jnp.tanh(ps.reshape(scenes, A, 1, EMB)
                      + pd.reshape(scenes, 1, A, EMB))
        m2 = jnp.tanh(jnp.dot(m1.reshape(scenes * A * A, EMB), msg_w2_ref[l],
                              preferred_element_type=jnp.float32)
                      + msg_b2_ref[l])
        aggr = m2.reshape(scenes, A, A, EMB).sum(axis=1).reshape(rows, EMB)
        wu = upd_w1_ref[l]
        u = jnp.tanh(jnp.dot(h, wu[:EMB], preferred_element_type=jnp.float32)
                     + jnp.dot(aggr, wu[EMB:], preferred_element_type=jnp.float32)
                     + upd_b1_ref[l])
        u = jnp.tanh(jnp.dot(u, upd_w2_ref[l], preferred_element_type=jnp.float32)
                     + upd_b2_ref[l])
        h = h + u

    pooled = h.reshape(scenes, A, EMB).sum(axis=1) * (1.0 / A)
    o = jnp.tanh(jnp.dot(pooled, fc_w1_ref[...], preferred_element_type=jnp.float32)
                 + fc_b1_ref[...])
    out_ref[...] = (jnp.dot(o, fc_w2_ref[...], preferred_element_type=jnp.float32)
                    + fc_b2_ref[...])


def kernel(pos, x_enc, pos_emb, numAgents_emb, T, W_in, b_in, msg_W1, msg_b1,
           msg_W2, msg_b2, upd_W1, upd_b1, upd_W2, upd_b2, fc_W1, fc_b1,
           fc_W2, fc_b2, *, interpret=False):
    del pos
    b, a = T.shape
    n = b * a
    x = jnp.concatenate([x_enc, pos_emb, numAgents_emb], axis=-1).reshape(n, -1)
    t = T.astype(jnp.int32).reshape(n, 1)

    scenes = SCENES_PER_BLOCK
    rows = scenes * a
    grid = (b // scenes,)

    def rowmap(i):
        return (i, 0)

    def fixed2(i):
        return (0, 0)

    def fixed3(i):
        return (0, 0, 0)

    full2 = lambda arr: pl.BlockSpec(arr.shape, fixed2)
    full3 = lambda arr: pl.BlockSpec(arr.shape, fixed3)

    b_in2 = b_in.reshape(1, EMB)
    msg_b1r = msg_b1.reshape(NUM_LAYERS, 1, EMB)
    msg_b2r = msg_b2.reshape(NUM_LAYERS, 1, EMB)
    upd_b1r = upd_b1.reshape(NUM_LAYERS, 1, EMB)
    upd_b2r = upd_b2.reshape(NUM_LAYERS, 1, EMB)
    fc_b1r = fc_b1.reshape(1, EMB // 2)
    fc_b2r = fc_b2.reshape(1, ENC_DIM)

    out = pl.pallas_call(
        functools.partial(_fwd_kernel, scenes=scenes),
        grid=grid,
        in_specs=[
            pl.BlockSpec((rows, x.shape[1]), rowmap),
            pl.BlockSpec((rows, 1), rowmap),
            full2(W_in), full2(b_in2),
            full3(msg_W1), full3(msg_b1r),
            full3(msg_W2), full3(msg_b2r),
            full3(upd_W1), full3(upd_b1r),
            full3(upd_W2), full3(upd_b2r),
            full2(fc_W1), full2(fc_b1r),
            full2(fc_W2), full2(fc_b2r),
        ],
        out_specs=pl.BlockSpec((scenes, ENC_DIM), rowmap),
        out_shape=jax.ShapeDtypeStruct((b, ENC_DIM), jnp.float32),
        compiler_params=pltpu.CompilerParams(
            dimension_semantics=("parallel",)),
        interpret=interpret,
    )(x, t, W_in, b_in2, msg_W1, msg_b1r, msg_W2, msg_b2r,
      upd_W1, upd_b1r, upd_W2, upd_b2r, fc_W1, fc_b1r, fc_W2, fc_b2r)
    return out
